# baseline (speedup 1.0000x reference)
import jax
import jax.numpy as jnp
from jax.experimental import pallas as pl
from jax.experimental.pallas import tpu as pltpu

_SOM_H = 64
_SOM_W = 64
_D = 512
_B = 4096
_N = _SOM_H * _SOM_W
_BN = 512
_NJ = _N // _BN


def _som_bmu_kernel(x_ref, wt_ref, idx_ref, row_ref, col_ref,
                    xsq_ref, minv_ref, mini_ref):
    j = pl.program_id(0)

    @pl.when(j == 0)
    def _init():
        xv = x_ref[...]
        xsq_ref[...] = jnp.sum(xv * xv, axis=1, keepdims=True)
        minv_ref[...] = jnp.full((_B, 1), jnp.inf, jnp.float32)
        mini_ref[...] = jnp.zeros((_B, 1), jnp.int32)

    wt = wt_ref[...]
    cross = jnp.dot(x_ref[...], wt, preferred_element_type=jnp.float32)
    w_sq = jnp.sum(wt * wt, axis=0, keepdims=True)
    dist = (xsq_ref[...] - 2.0 * cross) + w_sq

    tile_min = jnp.min(dist, axis=1, keepdims=True)
    lane = jax.lax.broadcasted_iota(jnp.int32, (_B, _BN), 1)
    tile_idx = jnp.min(
        jnp.where(dist == tile_min, lane, jnp.int32(_N)),
        axis=1, keepdims=True) + j * _BN

    better = tile_min < minv_ref[...]
    minv_ref[...] = jnp.where(better, tile_min, minv_ref[...])
    mini_ref[...] = jnp.where(better, tile_idx, mini_ref[...])

    @pl.when(j == _NJ - 1)
    def _done():
        bi = mini_ref[...]
        idx_ref[...] = bi
        row_ref[...] = bi // _SOM_W
        col_ref[...] = bi % _SOM_W


def kernel(x, weights):
    wt = weights.reshape(_N, _D).T
    idx, row, col = pl.pallas_call(
        _som_bmu_kernel,
        grid=(_NJ,),
        in_specs=[
            pl.BlockSpec((_B, _D), lambda j: (0, 0)),
            pl.BlockSpec((_D, _BN), lambda j: (0, j)),
        ],
        out_specs=[
            pl.BlockSpec((_B, 1), lambda j: (0, 0)),
            pl.BlockSpec((_B, 1), lambda j: (0, 0)),
            pl.BlockSpec((_B, 1), lambda j: (0, 0)),
        ],
        out_shape=[
            jax.ShapeDtypeStruct((_B, 1), jnp.int32),
            jax.ShapeDtypeStruct((_B, 1), jnp.int32),
            jax.ShapeDtypeStruct((_B, 1), jnp.int32),
        ],
        scratch_shapes=[
            pltpu.VMEM((_B, 1), jnp.float32),
            pltpu.VMEM((_B, 1), jnp.float32),
            pltpu.VMEM((_B, 1), jnp.int32),
        ],
        compiler_params=pltpu.CompilerParams(
            dimension_semantics=("arbitrary",)),
    )(x, wt)
    bmu_indices = idx[:, 0]
    bmu_coords = jnp.concatenate([row, col], axis=1)
    return bmu_coords, bmu_indices


# speedup vs baseline: 1.0862x; 1.0862x over previous
import jax
import jax.numpy as jnp
from jax.experimental import pallas as pl
from jax.experimental.pallas import tpu as pltpu

_SOM_H = 64
_SOM_W = 64
_D = 512
_B = 4096
_N = _SOM_H * _SOM_W
_BN = 512
_NJ = _N // _BN


def _som_bmu_kernel(x_ref, wt_ref, idx_ref, row_ref, col_ref,
                    xsq_ref, accv_ref, acct_ref):
    j = pl.program_id(0)

    @pl.when(j == 0)
    def _init():
        xv = x_ref[...]
        xsq_ref[...] = jnp.sum(xv * xv, axis=1, keepdims=True)
        accv_ref[...] = jnp.full((_B, _BN), jnp.inf, jnp.float32)
        acct_ref[...] = jnp.zeros((_B, _BN), jnp.int32)

    wt = wt_ref[...]
    cross = jnp.dot(x_ref[...], wt, preferred_element_type=jnp.float32)
    w_sq = jnp.sum(wt * wt, axis=0, keepdims=True)
    dist = (xsq_ref[...] - 2.0 * cross) + w_sq

    prev = accv_ref[...]
    better = dist < prev
    accv_ref[...] = jnp.where(better, dist, prev)
    acct_ref[...] = jnp.where(better, j, acct_ref[...])

    @pl.when(j == _NJ - 1)
    def _done():
        accv = accv_ref[...]
        lane = jax.lax.broadcasted_iota(jnp.int32, (_B, _BN), 1)
        flat = acct_ref[...] * _BN + lane
        fmin = jnp.min(accv, axis=1, keepdims=True)
        bi = jnp.min(jnp.where(accv == fmin, flat, jnp.int32(_N)),
                     axis=1, keepdims=True)
        idx_ref[...] = bi
        row_ref[...] = bi // _SOM_W
        col_ref[...] = bi % _SOM_W


def kernel(x, weights):
    wt = weights.reshape(_N, _D).T
    idx, row, col = pl.pallas_call(
        _som_bmu_kernel,
        grid=(_NJ,),
        in_specs=[
            pl.BlockSpec((_B, _D), lambda j: (0, 0)),
            pl.BlockSpec((_D, _BN), lambda j: (0, j)),
        ],
        out_specs=[
            pl.BlockSpec((_B, 1), lambda j: (0, 0)),
            pl.BlockSpec((_B, 1), lambda j: (0, 0)),
            pl.BlockSpec((_B, 1), lambda j: (0, 0)),
        ],
        out_shape=[
            jax.ShapeDtypeStruct((_B, 1), jnp.int32),
            jax.ShapeDtypeStruct((_B, 1), jnp.int32),
            jax.ShapeDtypeStruct((_B, 1), jnp.int32),
        ],
        scratch_shapes=[
            pltpu.VMEM((_B, 1), jnp.float32),
            pltpu.VMEM((_B, _BN), jnp.float32),
            pltpu.VMEM((_B, _BN), jnp.int32),
        ],
        compiler_params=pltpu.CompilerParams(
            dimension_semantics=("arbitrary",)),
    )(x, wt)
    bmu_indices = idx[:, 0]
    bmu_coords = jnp.concatenate([row, col], axis=1)
    return bmu_coords, bmu_indices


# speedup vs baseline: 1.8031x; 1.6600x over previous
import jax
import jax.numpy as jnp
from jax.experimental import pallas as pl
from jax.experimental.pallas import tpu as pltpu

_SOM_H = 64
_SOM_W = 64
_D = 512
_B = 4096
_N = _SOM_H * _SOM_W
_BN = 512
_NJ = _N // _BN

_DN = (((1,), (1,)), ((), ()))


def _som_bmu_kernel(x_ref, w_ref, idx_ref, row_ref, col_ref,
                    xsq_ref, accv_ref, acct_ref):
    j = pl.program_id(0)

    @pl.when(j == 0)
    def _init():
        xv = x_ref[...]
        xsq_ref[...] = jnp.sum(xv * xv, axis=1, keepdims=True)
        accv_ref[...] = jnp.full((_B, _BN), jnp.inf, jnp.float32)
        acct_ref[...] = jnp.zeros((_B, _BN), jnp.int32)

    w = w_ref[...]
    cross = jax.lax.dot_general(x_ref[...], w, _DN,
                                preferred_element_type=jnp.float32)
    wsq_col = jnp.sum(w * w, axis=1, keepdims=True)
    w_sq = jax.lax.transpose(wsq_col, (1, 0))
    dist = (xsq_ref[...] - 2.0 * cross) + w_sq

    prev = accv_ref[...]
    better = dist < prev
    accv_ref[...] = jnp.where(better, dist, prev)
    acct_ref[...] = jnp.where(better, j, acct_ref[...])

    @pl.when(j == _NJ - 1)
    def _done():
        accv = accv_ref[...]
        lane = jax.lax.broadcasted_iota(jnp.int32, (_B, _BN), 1)
        flat = acct_ref[...] * _BN + lane
        fmin = jnp.min(accv, axis=1, keepdims=True)
        bi = jnp.min(jnp.where(accv == fmin, flat, jnp.int32(_N)),
                     axis=1, keepdims=True)
        idx_ref[...] = bi
        row_ref[...] = bi // _SOM_W
        col_ref[...] = bi % _SOM_W


def kernel(x, weights):
    wf = weights.reshape(_N, _D)
    idx, row, col = pl.pallas_call(
        _som_bmu_kernel,
        grid=(_NJ,),
        in_specs=[
            pl.BlockSpec((_B, _D), lambda j: (0, 0)),
            pl.BlockSpec((_BN, _D), lambda j: (j, 0)),
        ],
        out_specs=[
            pl.BlockSpec((_B, 1), lambda j: (0, 0)),
            pl.BlockSpec((_B, 1), lambda j: (0, 0)),
            pl.BlockSpec((_B, 1), lambda j: (0, 0)),
        ],
        out_shape=[
            jax.ShapeDtypeStruct((_B, 1), jnp.int32),
            jax.ShapeDtypeStruct((_B, 1), jnp.int32),
            jax.ShapeDtypeStruct((_B, 1), jnp.int32),
        ],
        scratch_shapes=[
            pltpu.VMEM((_B, 1), jnp.float32),
            pltpu.VMEM((_B, _BN), jnp.float32),
            pltpu.VMEM((_B, _BN), jnp.int32),
        ],
        compiler_params=pltpu.CompilerParams(
            dimension_semantics=("arbitrary",)),
    )(x, wf)
    bmu_indices = idx[:, 0]
    bmu_coords = jnp.concatenate([row, col], axis=1)
    return bmu_coords, bmu_indices
